# baseline (speedup 1.0000x reference)
import functools

import jax
import jax.numpy as jnp
from jax import lax
from jax.experimental import pallas as pl
from jax.experimental.pallas import tpu as pltpu
from jax.experimental.pallas import tpu_sc as plsc

D_MODEL = 512
FEAT = 200
ROWS_PAD = 128

NUM_CORES = 2
NUM_SUBCORES = 16
NW = NUM_CORES * NUM_SUBCORES

CHUNK = 64


N_REP = 16


def _proj_body(table_ref, w_ref, b_ref, p_ref):
    p_ref[...] = (
        jnp.dot(table_ref[...], w_ref[...], preferred_element_type=jnp.float32)
        + b_ref[...]
    )


def _project_table(table_pad, W, b2):
    return pl.pallas_call(
        _proj_body,
        grid=(N_REP,),
        in_specs=[
            pl.BlockSpec((ROWS_PAD, FEAT), lambda k: (0, 0)),
            pl.BlockSpec((FEAT, D_MODEL), lambda k: (0, 0)),
            pl.BlockSpec((1, D_MODEL), lambda k: (0, 0)),
        ],
        out_specs=pl.BlockSpec((ROWS_PAD, D_MODEL), lambda k: (k, 0)),
        out_shape=jax.ShapeDtypeStruct((N_REP * ROWS_PAD, D_MODEL), jnp.float32),
    )(table_pad, W, b2)


def _make_gather(n_rows, t_dim):
    rows_per_w = n_rows // NW
    tok_per_w = rows_per_w * t_dim
    cpj = rows_per_w // CHUNK
    n_chunk = t_dim * cpj
    mesh = plsc.VectorSubcoreMesh(core_axis_name="c", subcore_axis_name="s")

    @functools.partial(
        pl.kernel,
        mesh=mesh,
        out_type=jax.ShapeDtypeStruct((t_dim, n_rows, D_MODEL), jnp.float32),
        scratch_types=[
            pltpu.VMEM((tok_per_w,), jnp.int32),
            pltpu.VMEM((CHUNK, D_MODEL), jnp.float32),
            pltpu.VMEM((CHUNK, D_MODEL), jnp.float32),
            pltpu.VMEM((CHUNK, D_MODEL), jnp.float32),
            pltpu.SemaphoreType.DMA,
            pltpu.SemaphoreType.DMA,
            pltpu.SemaphoreType.DMA,
            pltpu.SemaphoreType.DMA,
            pltpu.SemaphoreType.DMA,
            pltpu.SemaphoreType.DMA,
        ],
    )
    def gather_kernel(p_hbm, idx_hbm, out_hbm, idx_v, rows0, rows1, rows2,
                      gsem0, gsem1, gsem2, ssem0, ssem1, ssem2):
        wid = lax.axis_index("s") * NUM_CORES + lax.axis_index("c")
        row_base = wid * rows_per_w
        pltpu.sync_copy(idx_hbm.at[pl.ds(wid * tok_per_w, tok_per_w)], idx_v)

        rows = (rows0, rows1, rows2)
        gsem = (gsem0, gsem1, gsem2)
        ssem = (ssem0, ssem1, ssem2)

        def start_gather(m, b):
            pltpu.async_copy(
                p_hbm.at[idx_v.at[pl.ds(m * CHUNK, CHUNK)]], rows[b], gsem[b]
            )

        def start_scatter(m, b):
            t = m // cpj
            c = m - t * cpj
            pltpu.async_copy(
                rows[b], out_hbm.at[t, pl.ds(row_base + c * CHUNK, CHUNK)],
                ssem[b]
            )

        def wait_gather(b):
            pltpu.make_async_copy(p_hbm.at[idx_v.at[pl.ds(0, CHUNK)]],
                                  rows[b], gsem[b]).wait()

        def wait_scatter(b):
            pltpu.make_async_copy(rows[b], out_hbm.at[0, pl.ds(row_base, CHUNK)],
                                  ssem[b]).wait()

        start_gather(0, 0)
        start_gather(1, 1)
        n_grp = (n_chunk + 2) // 3

        def grp_body(i3, carry):
            for b in range(3):
                m = 3 * i3 + b
                nb = (b + 2) % 3

                @pl.when(m < n_chunk)
                def _():
                    wait_gather(b)

                    @pl.when(m + 2 < n_chunk)
                    def _():
                        @pl.when(m >= 1)
                        def _():
                            wait_scatter(nb)

                        start_gather(m + 2, nb)

                    start_scatter(m, b)
            return carry

        lax.fori_loop(0, n_grp, grp_body, 0)
        wait_scatter((n_chunk - 1) % 3)
        wait_scatter((n_chunk - 2) % 3)
        wait_scatter((n_chunk - 3) % 3)

    return gather_kernel


def kernel(src, table, W, b):
    B, T = src.shape
    table_pad = jnp.pad(table, ((0, ROWS_PAD - table.shape[0]), (0, 0)))
    P = _project_table(table_pad, W, b.reshape(1, D_MODEL))
    idx3 = jnp.transpose(src.reshape(NW, B // NW, T), (0, 2, 1))
    rep_off = (jnp.arange(NW, dtype=jnp.int32) % N_REP) * ROWS_PAD
    idx = (idx3 + rep_off[:, None, None]).reshape(-1)
    out_t = _make_gather(B, T)(P, idx)
    return jnp.transpose(out_t, (1, 0, 2))


# speedup vs baseline: 1.0361x; 1.0361x over previous
import functools

import jax
import jax.numpy as jnp
from jax import lax
from jax.experimental import pallas as pl
from jax.experimental.pallas import tpu as pltpu
from jax.experimental.pallas import tpu_sc as plsc

D_MODEL = 512
FEAT = 200
ROWS_PAD = 128

NUM_CORES = 2
NUM_SUBCORES = 16
NW = NUM_CORES * NUM_SUBCORES

CHUNK = 64


N_REP = 64


def _proj_body(table_ref, w_ref, b_ref, p_ref):
    p_ref[...] = (
        jnp.dot(table_ref[...], w_ref[...], preferred_element_type=jnp.float32)
        + b_ref[...]
    )


def _project_table(table_pad, W, b2):
    return pl.pallas_call(
        _proj_body,
        grid=(N_REP,),
        in_specs=[
            pl.BlockSpec((ROWS_PAD, FEAT), lambda k: (0, 0)),
            pl.BlockSpec((FEAT, D_MODEL), lambda k: (0, 0)),
            pl.BlockSpec((1, D_MODEL), lambda k: (0, 0)),
        ],
        out_specs=pl.BlockSpec((ROWS_PAD, D_MODEL), lambda k: (k, 0)),
        out_shape=jax.ShapeDtypeStruct((N_REP * ROWS_PAD, D_MODEL), jnp.float32),
    )(table_pad, W, b2)


def _make_gather(n_rows, t_dim):
    rows_per_w = n_rows // NW
    tok_per_w = rows_per_w * t_dim
    cpj = rows_per_w // CHUNK
    n_chunk = t_dim * cpj
    mesh = plsc.VectorSubcoreMesh(core_axis_name="c", subcore_axis_name="s")

    @functools.partial(
        pl.kernel,
        mesh=mesh,
        out_type=jax.ShapeDtypeStruct((t_dim, n_rows, D_MODEL), jnp.float32),
        scratch_types=[
            pltpu.VMEM((tok_per_w,), jnp.int32),
            pltpu.VMEM((CHUNK, D_MODEL), jnp.float32),
            pltpu.VMEM((CHUNK, D_MODEL), jnp.float32),
            pltpu.VMEM((CHUNK, D_MODEL), jnp.float32),
            pltpu.SemaphoreType.DMA,
            pltpu.SemaphoreType.DMA,
            pltpu.SemaphoreType.DMA,
            pltpu.SemaphoreType.DMA,
            pltpu.SemaphoreType.DMA,
            pltpu.SemaphoreType.DMA,
        ],
    )
    def gather_kernel(p_hbm, idx_hbm, out_hbm, idx_v, rows0, rows1, rows2,
                      gsem0, gsem1, gsem2, ssem0, ssem1, ssem2):
        wid = lax.axis_index("s") * NUM_CORES + lax.axis_index("c")
        row_base = wid * rows_per_w
        pltpu.sync_copy(idx_hbm.at[pl.ds(wid * tok_per_w, tok_per_w)], idx_v)

        rows = (rows0, rows1, rows2)
        gsem = (gsem0, gsem1, gsem2)
        ssem = (ssem0, ssem1, ssem2)

        def start_gather(m, b):
            pltpu.async_copy(
                p_hbm.at[idx_v.at[pl.ds(m * CHUNK, CHUNK)]], rows[b], gsem[b]
            )

        def start_scatter(m, b):
            t = m // cpj
            c = m - t * cpj
            pltpu.async_copy(
                rows[b], out_hbm.at[t, pl.ds(row_base + c * CHUNK, CHUNK)],
                ssem[b]
            )

        def wait_gather(b):
            pltpu.make_async_copy(p_hbm.at[idx_v.at[pl.ds(0, CHUNK)]],
                                  rows[b], gsem[b]).wait()

        def wait_scatter(b):
            pltpu.make_async_copy(rows[b], out_hbm.at[0, pl.ds(row_base, CHUNK)],
                                  ssem[b]).wait()

        start_gather(0, 0)
        start_gather(1, 1)
        n_grp = (n_chunk + 2) // 3

        def grp_body(i3, carry):
            for b in range(3):
                m = 3 * i3 + b
                nb = (b + 2) % 3

                @pl.when(m < n_chunk)
                def _():
                    wait_gather(b)

                    @pl.when(m + 2 < n_chunk)
                    def _():
                        @pl.when(m >= 1)
                        def _():
                            wait_scatter(nb)

                        start_gather(m + 2, nb)

                    start_scatter(m, b)
            return carry

        lax.fori_loop(0, n_grp, grp_body, 0)
        wait_scatter((n_chunk - 1) % 3)
        wait_scatter((n_chunk - 2) % 3)
        wait_scatter((n_chunk - 3) % 3)

    return gather_kernel


def kernel(src, table, W, b):
    B, T = src.shape
    table_pad = jnp.pad(table, ((0, ROWS_PAD - table.shape[0]), (0, 0)))
    P = _project_table(table_pad, W, b.reshape(1, D_MODEL))
    idx3 = jnp.transpose(src.reshape(NW, B // NW, T), (0, 2, 1))
    k_per_w = N_REP // NW
    w_off = jnp.arange(NW, dtype=jnp.int32) * (k_per_w * ROWS_PAD)
    par_off = (jnp.arange(B // NW, dtype=jnp.int32) % k_per_w) * ROWS_PAD
    idx = (idx3 + w_off[:, None, None] + par_off[None, None, :]).reshape(-1)
    out_t = _make_gather(B, T)(P, idx)
    return jnp.transpose(out_t, (1, 0, 2))
